# Initial kernel scaffold; baseline (speedup 1.0000x reference)
import functools

import jax
import jax.numpy as jnp
from jax import lax
from jax.experimental import pallas as pl
from jax.experimental.pallas import tpu as pltpu
from jax.experimental.pallas import tpu_sc as plsc

N = 10000
E = 160000
R = 200
ENT = 256
REL = 128
TIME = 64
IN_MSG = TIME + REL + ENT

NC = 2
NS = 16
CH = 128
EP = 163840
KA = 20480
KB = 416
SA_COLS = 512
F32 = jnp.float32


def _sc_stage1(eq0, eq1, eq2, eq3, rh0, rh1, t64, ones_t, zeros_t,
               srcP, dstP, brelP, timeP, invP):
    mesh = plsc.VectorSubcoreMesh(core_axis_name="c", subcore_axis_name="s")

    @functools.partial(
        pl.kernel,
        out_type=jax.ShapeDtypeStruct((2 * N, SA_COLS), F32),
        mesh=mesh,
        scratch_types=[
            pltpu.VMEM_SHARED((KA, 64), F32),
            pltpu.VMEM((CH,), jnp.int32),
            pltpu.VMEM((CH,), jnp.int32),
            pltpu.VMEM((CH,), jnp.int32),
            pltpu.VMEM((CH,), jnp.int32),
            pltpu.VMEM((CH, 64), F32),
            pltpu.VMEM((CH, 64), F32),
            pltpu.VMEM((CH, 64), F32),
            pltpu.SemaphoreType.DMA,
        ],
    )
    def k(eq0_h, eq1_h, eq2_h, eq3_h, rh0_h, rh1_h, t64_h, ones_h, zeros_h,
          src_h, dst_h, brel_h, time_h, inv_h, out_h,
          acc, gidx, dstv, invv, keyv, rows, rones, zbuf, sem):
        core = lax.axis_index("c")
        s = lax.axis_index("s")
        pltpu.sync_copy(ones_h, rones)
        pltpu.sync_copy(zeros_h, zbuf)
        eps = EP // NS
        nch = eps // CH

        def run_pass(cid, table, idx_src, gather):
            @pl.when(core == cid)
            def _():
                def body(i, carry):
                    base = s * eps + i * CH
                    pltpu.sync_copy(dst_h.at[pl.ds(base, CH)], dstv)
                    pltpu.sync_copy(inv_h.at[pl.ds(base, CH)], invv)
                    for j in range(CH // 16):
                        sl = pl.ds(j * 16, 16)
                        keyv[sl] = invv[sl] * N + dstv[sl]
                    if gather:
                        pltpu.sync_copy(idx_src.at[pl.ds(base, CH)], gidx)
                        pltpu.async_copy(table.at[gidx], rows, sem).wait()
                        pltpu.sync_copy(rows, acc.at[keyv], add=True)
                    else:
                        pltpu.sync_copy(rones, acc.at[keyv], add=True)
                    return carry
                lax.fori_loop(0, nch, body, 0)

        def flush(cid, coff):
            @pl.when(core == cid)
            def _():
                nr = 2 * N // NS
                r0 = s * nr
                pltpu.sync_copy(acc.at[pl.ds(r0, nr)],
                                out_h.at[pl.ds(r0, nr), pl.ds(coff, 64)])

        rounds = [
            ((eq0_h, src_h, 128, True), (eq1_h, src_h, 192, True)),
            ((eq2_h, src_h, 256, True), (eq3_h, src_h, 320, True)),
            ((rh0_h, brel_h, 0, True), (rh1_h, brel_h, 64, True)),
            ((t64_h, time_h, 384, True), (None, None, 448, False)),
        ]
        for p0, p1 in rounds:
            for z in range(KA // NS // CH):
                pltpu.sync_copy(zbuf, acc.at[pl.ds(s * (KA // NS) + z * CH, CH)])
            plsc.subcore_barrier()
            run_pass(0, p0[0], p0[1], p0[3])
            run_pass(1, p1[0], p1[1], p1[3])
            plsc.subcore_barrier()
            flush(0, p0[2])
            flush(1, p1[2])
            plsc.subcore_barrier()

    return k(eq0, eq1, eq2, eq3, rh0, rh1, t64, ones_t, zeros_t,
             srcP, dstP, brelP, timeP, invP)


def _sc_stage2(ent_new, t64, ones_t, zeros_t, srcP, dstP, brelP, timeP, invP):
    mesh = plsc.VectorSubcoreMesh(core_axis_name="c", subcore_axis_name="s")

    @functools.partial(
        pl.kernel,
        out_type=(
            jax.ShapeDtypeStruct((NC, KB, ENT), F32),
            jax.ShapeDtypeStruct((NC, KB, ENT), F32),
            jax.ShapeDtypeStruct((NC, KB, 64), F32),
            jax.ShapeDtypeStruct((NC, KB, 64), F32),
        ),
        mesh=mesh,
        scratch_types=[
            pltpu.VMEM_SHARED((KB, ENT), F32),
            pltpu.VMEM_SHARED((KB, ENT), F32),
            pltpu.VMEM_SHARED((KB, 64), F32),
            pltpu.VMEM_SHARED((KB, 64), F32),
            pltpu.VMEM((CH,), jnp.int32),
            pltpu.VMEM((CH,), jnp.int32),
            pltpu.VMEM((CH,), jnp.int32),
            pltpu.VMEM((CH,), jnp.int32),
            pltpu.VMEM((CH, ENT), F32),
            pltpu.VMEM((CH, ENT), F32),
            pltpu.VMEM((CH, 64), F32),
            pltpu.VMEM((CH, 64), F32),
            pltpu.VMEM((104, ENT), F32),
            pltpu.SemaphoreType.DMA,
        ],
    )
    def k(ent_h, t64_h, ones_h, zeros_h, src_h, dst_h, brel_h, time_h, inv_h,
          outS_h, outD_h, outT_h, outC_h,
          accS, accD, accT, accC, gidx, brelv, invv, keyv,
          rowsS, rowsD, rowsT, rones, zbuf, sem):
        core = lax.axis_index("c")
        s = lax.axis_index("s")
        pltpu.sync_copy(ones_h, rones)
        pltpu.sync_copy(zeros_h, zbuf)

        @pl.when(s < 4)
        def _():
            pltpu.sync_copy(zbuf, accS.at[pl.ds(s * 104, 104)])

        @pl.when((s >= 4) & (s < 8))
        def _():
            pltpu.sync_copy(zbuf, accD.at[pl.ds((s - 4) * 104, 104)])

        @pl.when((s >= 8) & (s < 12))
        def _():
            pltpu.sync_copy(zbuf.at[pl.ds(0, 104), pl.ds(0, 64)],
                            accT.at[pl.ds((s - 8) * 104, 104)])

        @pl.when(s >= 12)
        def _():
            pltpu.sync_copy(zbuf.at[pl.ds(0, 104), pl.ds(0, 64)],
                            accC.at[pl.ds((s - 12) * 104, 104)])

        plsc.subcore_barrier()

        wid = s * NC + core
        eps = EP // (NC * NS)
        nch = eps // CH

        def body(i, carry):
            base = wid * eps + i * CH
            pltpu.sync_copy(brel_h.at[pl.ds(base, CH)], brelv)
            pltpu.sync_copy(inv_h.at[pl.ds(base, CH)], invv)
            for j in range(CH // 16):
                sl = pl.ds(j * 16, 16)
                keyv[sl] = invv[sl] * R + brelv[sl]
            pltpu.sync_copy(src_h.at[pl.ds(base, CH)], gidx)
            pltpu.async_copy(ent_h.at[gidx], rowsS, sem).wait()
            pltpu.sync_copy(rowsS, accS.at[keyv], add=True)
            pltpu.sync_copy(dst_h.at[pl.ds(base, CH)], gidx)
            pltpu.async_copy(ent_h.at[gidx], rowsD, sem).wait()
            pltpu.sync_copy(rowsD, accD.at[keyv], add=True)
            pltpu.sync_copy(time_h.at[pl.ds(base, CH)], gidx)
            pltpu.async_copy(t64_h.at[gidx], rowsT, sem).wait()
            pltpu.sync_copy(rowsT, accT.at[keyv], add=True)
            pltpu.sync_copy(rones, accC.at[keyv], add=True)
            return carry

        lax.fori_loop(0, nch, body, 0)
        plsc.subcore_barrier()

        nr = KB // NS
        r0 = s * nr
        pltpu.sync_copy(accS.at[pl.ds(r0, nr)], outS_h.at[core, pl.ds(r0, nr)])
        pltpu.sync_copy(accD.at[pl.ds(r0, nr)], outD_h.at[core, pl.ds(r0, nr)])
        pltpu.sync_copy(accT.at[pl.ds(r0, nr)], outT_h.at[core, pl.ds(r0, nr)])
        pltpu.sync_copy(accC.at[pl.ds(r0, nr)], outC_h.at[core, pl.ds(r0, nr)])

    return k(ent_new, t64, ones_t, zeros_t, srcP, dstP, brelP, timeP, invP)


def _tc_stage1(SA, ent_emb, wIt, wOt, wSt, bias3):
    BM = 1000
    nb = N // BM

    def body(s0_ref, s1_ref, e_ref, wI_ref, wO_ref, wS_ref, b_ref, o_ref):
        dot = functools.partial(jnp.dot, preferred_element_type=F32,
                                precision=lax.Precision.HIGHEST)
        blk0 = s0_ref[...]
        blk1 = s1_ref[...]
        s0 = blk0[:, :IN_MSG]
        d0 = blk0[:, IN_MSG:IN_MSG + 1]
        s1 = blk1[:, :IN_MSG]
        d1 = blk1[:, IN_MSG:IN_MSG + 1]
        m = (dot(s0, wI_ref[...]) + d0 * b_ref[0:1, :]
             + dot(s1, wO_ref[...]) + d1 * b_ref[1:2, :])
        h = m / jnp.maximum(d0 + d1, 1.0)
        o_ref[...] = dot(e_ref[...], wS_ref[...]) + b_ref[2:3, :] + h

    return pl.pallas_call(
        body,
        grid=(nb,),
        in_specs=[
            pl.BlockSpec((BM, SA_COLS), lambda i: (i, 0)),
            pl.BlockSpec((BM, SA_COLS), lambda i: (i + nb, 0)),
            pl.BlockSpec((BM, ENT), lambda i: (i, 0)),
            pl.BlockSpec((IN_MSG, ENT), lambda i: (0, 0)),
            pl.BlockSpec((IN_MSG, ENT), lambda i: (0, 0)),
            pl.BlockSpec((ENT, ENT), lambda i: (0, 0)),
            pl.BlockSpec((8, ENT), lambda i: (0, 0)),
        ],
        out_specs=pl.BlockSpec((BM, ENT), lambda i: (i, 0)),
        out_shape=jax.ShapeDtypeStruct((N, ENT), F32),
    )(SA, SA, ent_emb, wIt, wOt, wSt, bias3)


def _tc_stage2(outS, outD, outT, outC, rel_emb, wot, wit, wrt, bias3r):
    def body(S_ref, D_ref, T_ref, C_ref, rel_ref, wo_ref, wi_ref, wr_ref,
             b_ref, o_ref):
        dot = functools.partial(jnp.dot, preferred_element_type=F32,
                                precision=lax.Precision.HIGHEST)
        US = S_ref[0] + S_ref[1]
        UD = D_ref[0] + D_ref[1]
        UT = T_ref[0] + T_ref[1]
        Cc = C_ref[0] + C_ref[1]
        c = Cc[:, 0:1]
        p0 = (dot(US[0:R], wo_ref[0:ENT]) + dot(UD[0:R], wo_ref[ENT:2 * ENT])
              + dot(UT[0:R], wo_ref[2 * ENT:2 * ENT + TIME])
              + c[0:R] * b_ref[0:1, :])
        p1 = (dot(US[R:2 * R], wi_ref[0:ENT])
              + dot(UD[R:2 * R], wi_ref[ENT:2 * ENT])
              + dot(UT[R:2 * R], wi_ref[2 * ENT:2 * ENT + TIME])
              + c[R:2 * R] * b_ref[1:2, :])
        cnt = c[0:R] + c[R:2 * R]
        h = (p0 + p1) / jnp.maximum(cnt, 1.0)
        o_ref[...] = dot(rel_ref[...], wr_ref[...]) + b_ref[2:3, :] + h

    return pl.pallas_call(
        body,
        out_shape=jax.ShapeDtypeStruct((R, REL), F32),
    )(outS, outD, outT, outC, rel_emb, wot, wit, wrt, bias3r)


def kernel(ent_emb, rel_emb, time_emb, edge_index, b_rel, time_idx, inv,
           W_I_w, W_I_b, W_O_w, W_O_b, W_S_w, W_S_b,
           W_r_ori_w, W_r_ori_b, W_r_inv_w, W_r_inv_b, W_R_w, W_R_b):
    i32 = jnp.int32
    pad = EP - E
    src = edge_index[0].astype(i32)
    dst = edge_index[1].astype(i32)
    zpad = jnp.zeros((pad,), i32)
    srcP = jnp.concatenate([src, zpad])
    dstP = jnp.concatenate([dst, zpad])
    brelP = jnp.concatenate([b_rel.astype(i32), zpad])
    timeP = jnp.concatenate([time_idx.astype(i32), zpad])
    invP = jnp.concatenate([inv.astype(i32), jnp.full((pad,), 2, i32)])

    eq = [ent_emb[:, 64 * k:64 * (k + 1)] for k in range(4)]
    rh0 = rel_emb[:, :64]
    rh1 = rel_emb[:, 64:]
    ones_t = jnp.ones((CH, 64), F32)
    zeros_t = jnp.zeros((CH, 64), F32)

    SA = _sc_stage1(eq[0], eq[1], eq[2], eq[3], rh0, rh1, time_emb,
                    ones_t, zeros_t, srcP, dstP, brelP, timeP, invP)

    bias3 = jnp.concatenate(
        [W_I_b[None], W_O_b[None], W_S_b[None], jnp.zeros((5, ENT), F32)], 0)
    ent_new = _tc_stage1(SA, ent_emb, W_I_w.T, W_O_w.T, W_S_w.T, bias3)

    zeros_t2 = jnp.zeros((104, ENT), F32)
    outS, outD, outT, outC = _sc_stage2(
        ent_new, time_emb, ones_t, zeros_t2, srcP, dstP, brelP, timeP, invP)

    bias3r = jnp.concatenate(
        [W_r_ori_b[None], W_r_inv_b[None], W_R_b[None], jnp.zeros((5, REL), F32)], 0)
    rel_new = _tc_stage2(outS, outD, outT, outC, rel_emb,
                         W_r_ori_w.T, W_r_inv_w.T, W_R_w.T, bias3r)
    return ent_new, rel_new


# speedup vs baseline: 1.7702x; 1.7702x over previous
import functools

import jax
import jax.numpy as jnp
from jax import lax
from jax.experimental import pallas as pl
from jax.experimental.pallas import tpu as pltpu
from jax.experimental.pallas import tpu_sc as plsc

N = 10000
E = 160000
R = 200
ENT = 256
REL = 128
TIME = 64
IN_MSG = TIME + REL + ENT

NC = 2
NS = 16
CH = 128
EP = 163840
KA = 20480
KB = 416
F32 = jnp.float32
NPLANES = 8


def _sc_stage1(eq0, eq1, eq2, eq3, rh0, rh1, t64, ones_t, zeros_t,
               srcP, dstP, brelP, timeP, invP):
    mesh = plsc.VectorSubcoreMesh(core_axis_name="c", subcore_axis_name="s")

    @functools.partial(
        pl.kernel,
        out_type=jax.ShapeDtypeStruct((NPLANES, KA, 64), F32),
        mesh=mesh,
        compiler_params=pltpu.CompilerParams(use_tc_tiling_on_sc=False),
        scratch_types=[
            pltpu.VMEM_SHARED((KA, 64), F32),
            pltpu.VMEM((CH,), jnp.int32),
            pltpu.VMEM((CH,), jnp.int32),
            pltpu.VMEM((CH,), jnp.int32),
            pltpu.VMEM((CH,), jnp.int32),
            pltpu.VMEM((CH, 64), F32),
            pltpu.VMEM((CH, 64), F32),
            pltpu.VMEM((CH, 64), F32),
            pltpu.SemaphoreType.DMA,
        ],
    )
    def k(eq0_h, eq1_h, eq2_h, eq3_h, rh0_h, rh1_h, t64_h, ones_h, zeros_h,
          src_h, dst_h, brel_h, time_h, inv_h, out_h,
          acc, gidx, dstv, invv, keyv, rows, rones, zbuf, sem):
        core = lax.axis_index("c")
        s = lax.axis_index("s")
        pltpu.sync_copy(ones_h, rones)
        pltpu.sync_copy(zeros_h, zbuf)
        eps = EP // NS
        nch = eps // CH

        def run_pass(cid, table, idx_src, gather):
            @pl.when(core == cid)
            def _():
                def body(i, carry):
                    base = s * eps + i * CH
                    pltpu.sync_copy(dst_h.at[pl.ds(base, CH)], dstv)
                    pltpu.sync_copy(inv_h.at[pl.ds(base, CH)], invv)
                    for j in range(CH // 16):
                        sl = pl.ds(j * 16, 16)
                        keyv[sl] = invv[sl] * N + dstv[sl]
                    if gather:
                        pltpu.sync_copy(idx_src.at[pl.ds(base, CH)], gidx)
                        pltpu.async_copy(table.at[gidx], rows, sem).wait()
                        pltpu.sync_copy(rows, acc.at[keyv], add=True)
                    else:
                        pltpu.sync_copy(rones, acc.at[keyv], add=True)
                    return carry
                lax.fori_loop(0, nch, body, 0)

        def flush(cid, plane):
            @pl.when(core == cid)
            def _():
                nr = KA // NS
                r0 = s * nr
                pltpu.sync_copy(acc.at[pl.ds(r0, nr)],
                                out_h.at[plane, pl.ds(r0, nr)])

        rounds = [
            ((eq0_h, src_h, 2, True), (eq1_h, src_h, 3, True)),
            ((eq2_h, src_h, 4, True), (eq3_h, src_h, 5, True)),
            ((rh0_h, brel_h, 0, True), (rh1_h, brel_h, 1, True)),
            ((t64_h, time_h, 6, True), (None, None, 7, False)),
        ]
        for p0, p1 in rounds:
            for z in range(KA // NS // CH):
                pltpu.sync_copy(zbuf, acc.at[pl.ds(s * (KA // NS) + z * CH, CH)])
            plsc.subcore_barrier()
            run_pass(0, p0[0], p0[1], p0[3])
            run_pass(1, p1[0], p1[1], p1[3])
            plsc.subcore_barrier()
            flush(0, p0[2])
            flush(1, p1[2])
            plsc.subcore_barrier()

    return k(eq0, eq1, eq2, eq3, rh0, rh1, t64, ones_t, zeros_t,
             srcP, dstP, brelP, timeP, invP)


def _sc_stage2(ent_new, t64, ones_t, zeros_t, srcP, dstP, brelP, timeP, invP):
    mesh = plsc.VectorSubcoreMesh(core_axis_name="c", subcore_axis_name="s")

    @functools.partial(
        pl.kernel,
        out_type=(
            jax.ShapeDtypeStruct((NC, KB, ENT), F32),
            jax.ShapeDtypeStruct((NC, KB, ENT), F32),
            jax.ShapeDtypeStruct((NC, KB, 64), F32),
            jax.ShapeDtypeStruct((NC, KB, 64), F32),
        ),
        mesh=mesh,
        compiler_params=pltpu.CompilerParams(use_tc_tiling_on_sc=False),
        scratch_types=[
            pltpu.VMEM_SHARED((KB, ENT), F32),
            pltpu.VMEM_SHARED((KB, ENT), F32),
            pltpu.VMEM_SHARED((KB, 64), F32),
            pltpu.VMEM_SHARED((KB, 64), F32),
            pltpu.VMEM((CH,), jnp.int32),
            pltpu.VMEM((CH,), jnp.int32),
            pltpu.VMEM((CH,), jnp.int32),
            pltpu.VMEM((CH,), jnp.int32),
            pltpu.VMEM((CH, ENT), F32),
            pltpu.VMEM((CH, ENT), F32),
            pltpu.VMEM((CH, 64), F32),
            pltpu.VMEM((CH, 64), F32),
            pltpu.VMEM((104, ENT), F32),
            pltpu.SemaphoreType.DMA,
        ],
    )
    def k(ent_h, t64_h, ones_h, zeros_h, src_h, dst_h, brel_h, time_h, inv_h,
          outS_h, outD_h, outT_h, outC_h,
          accS, accD, accT, accC, gidx, brelv, invv, keyv,
          rowsS, rowsD, rowsT, rones, zbuf, sem):
        core = lax.axis_index("c")
        s = lax.axis_index("s")
        pltpu.sync_copy(ones_h, rones)
        pltpu.sync_copy(zeros_h, zbuf)

        @pl.when(s < 4)
        def _():
            pltpu.sync_copy(zbuf, accS.at[pl.ds(s * 104, 104)])

        @pl.when((s >= 4) & (s < 8))
        def _():
            pltpu.sync_copy(zbuf, accD.at[pl.ds((s - 4) * 104, 104)])

        @pl.when((s >= 8) & (s < 12))
        def _():
            pltpu.sync_copy(zbuf.at[pl.ds(0, 104), pl.ds(0, 64)],
                            accT.at[pl.ds((s - 8) * 104, 104)])

        @pl.when(s >= 12)
        def _():
            pltpu.sync_copy(zbuf.at[pl.ds(0, 104), pl.ds(0, 64)],
                            accC.at[pl.ds((s - 12) * 104, 104)])

        plsc.subcore_barrier()

        wid = s * NC + core
        eps = EP // (NC * NS)
        nch = eps // CH

        def body(i, carry):
            base = wid * eps + i * CH
            pltpu.sync_copy(brel_h.at[pl.ds(base, CH)], brelv)
            pltpu.sync_copy(inv_h.at[pl.ds(base, CH)], invv)
            for j in range(CH // 16):
                sl = pl.ds(j * 16, 16)
                keyv[sl] = invv[sl] * R + brelv[sl]
            pltpu.sync_copy(src_h.at[pl.ds(base, CH)], gidx)
            pltpu.async_copy(ent_h.at[gidx], rowsS, sem).wait()
            pltpu.sync_copy(rowsS, accS.at[keyv], add=True)
            pltpu.sync_copy(dst_h.at[pl.ds(base, CH)], gidx)
            pltpu.async_copy(ent_h.at[gidx], rowsD, sem).wait()
            pltpu.sync_copy(rowsD, accD.at[keyv], add=True)
            pltpu.sync_copy(time_h.at[pl.ds(base, CH)], gidx)
            pltpu.async_copy(t64_h.at[gidx], rowsT, sem).wait()
            pltpu.sync_copy(rowsT, accT.at[keyv], add=True)
            pltpu.sync_copy(rones, accC.at[keyv], add=True)
            return carry

        lax.fori_loop(0, nch, body, 0)
        plsc.subcore_barrier()

        @pl.when(s < 13)
        def _():
            nr = 32
            r0 = s * nr
            pltpu.sync_copy(accS.at[pl.ds(r0, nr)], outS_h.at[core, pl.ds(r0, nr)])
            pltpu.sync_copy(accD.at[pl.ds(r0, nr)], outD_h.at[core, pl.ds(r0, nr)])
            pltpu.sync_copy(accT.at[pl.ds(r0, nr)], outT_h.at[core, pl.ds(r0, nr)])
            pltpu.sync_copy(accC.at[pl.ds(r0, nr)], outC_h.at[core, pl.ds(r0, nr)])

    return k(ent_new, t64, ones_t, zeros_t, srcP, dstP, brelP, timeP, invP)


def _tc_stage1(SA, ent_emb, wIt, wOt, wSt, bias3):
    BM = 1000
    nb = N // BM

    def body(s0_ref, s1_ref, e_ref, wI_ref, wO_ref, wS_ref, b_ref, o_ref):
        dot = functools.partial(jnp.dot, preferred_element_type=F32,
                                precision=lax.Precision.HIGHEST)
        blk0 = s0_ref[...]
        blk1 = s1_ref[...]
        s0 = jnp.concatenate([blk0[p] for p in range(7)], axis=1)
        d0 = blk0[7][:, 0:1]
        s1 = jnp.concatenate([blk1[p] for p in range(7)], axis=1)
        d1 = blk1[7][:, 0:1]
        m = (dot(s0, wI_ref[...]) + d0 * b_ref[0:1, :]
             + dot(s1, wO_ref[...]) + d1 * b_ref[1:2, :])
        h = m / jnp.maximum(d0 + d1, 1.0)
        o_ref[...] = dot(e_ref[...], wS_ref[...]) + b_ref[2:3, :] + h

    return pl.pallas_call(
        body,
        grid=(nb,),
        in_specs=[
            pl.BlockSpec((NPLANES, BM, 64), lambda i: (0, i, 0)),
            pl.BlockSpec((NPLANES, BM, 64), lambda i: (0, i + nb, 0)),
            pl.BlockSpec((BM, ENT), lambda i: (i, 0)),
            pl.BlockSpec((IN_MSG, ENT), lambda i: (0, 0)),
            pl.BlockSpec((IN_MSG, ENT), lambda i: (0, 0)),
            pl.BlockSpec((ENT, ENT), lambda i: (0, 0)),
            pl.BlockSpec((8, ENT), lambda i: (0, 0)),
        ],
        out_specs=pl.BlockSpec((BM, ENT), lambda i: (i, 0)),
        out_shape=jax.ShapeDtypeStruct((N, ENT), F32),
    )(SA, SA, ent_emb, wIt, wOt, wSt, bias3)


def _tc_stage2(outS, outD, outT, outC, rel_emb, wot, wit, wrt, bias3r):
    def body(S_ref, D_ref, T_ref, C_ref, rel_ref, wo_ref, wi_ref, wr_ref,
             b_ref, o_ref):
        dot = functools.partial(jnp.dot, preferred_element_type=F32,
                                precision=lax.Precision.HIGHEST)
        US = S_ref[0] + S_ref[1]
        UD = D_ref[0] + D_ref[1]
        UT = T_ref[0] + T_ref[1]
        Cc = C_ref[0] + C_ref[1]
        c = Cc[:, 0:1]
        p0 = (dot(US[0:R], wo_ref[0:ENT]) + dot(UD[0:R], wo_ref[ENT:2 * ENT])
              + dot(UT[0:R], wo_ref[2 * ENT:2 * ENT + TIME])
              + c[0:R] * b_ref[0:1, :])
        p1 = (dot(US[R:2 * R], wi_ref[0:ENT])
              + dot(UD[R:2 * R], wi_ref[ENT:2 * ENT])
              + dot(UT[R:2 * R], wi_ref[2 * ENT:2 * ENT + TIME])
              + c[R:2 * R] * b_ref[1:2, :])
        cnt = c[0:R] + c[R:2 * R]
        h = (p0 + p1) / jnp.maximum(cnt, 1.0)
        o_ref[...] = dot(rel_ref[...], wr_ref[...]) + b_ref[2:3, :] + h

    return pl.pallas_call(
        body,
        out_shape=jax.ShapeDtypeStruct((R, REL), F32),
    )(outS, outD, outT, outC, rel_emb, wot, wit, wrt, bias3r)


def kernel(ent_emb, rel_emb, time_emb, edge_index, b_rel, time_idx, inv,
           W_I_w, W_I_b, W_O_w, W_O_b, W_S_w, W_S_b,
           W_r_ori_w, W_r_ori_b, W_r_inv_w, W_r_inv_b, W_R_w, W_R_b):
    i32 = jnp.int32
    pad = EP - E
    src = edge_index[0].astype(i32)
    dst = edge_index[1].astype(i32)
    zpad = jnp.zeros((pad,), i32)
    srcP = jnp.concatenate([src, zpad])
    dstP = jnp.concatenate([dst, zpad])
    brelP = jnp.concatenate([b_rel.astype(i32), zpad])
    timeP = jnp.concatenate([time_idx.astype(i32), zpad])
    invP = jnp.concatenate([inv.astype(i32), jnp.full((pad,), 2, i32)])

    eq = [ent_emb[:, 64 * k:64 * (k + 1)] for k in range(4)]
    rh0 = rel_emb[:, :64]
    rh1 = rel_emb[:, 64:]
    ones_t = jnp.ones((CH, 64), F32)
    zeros_t = jnp.zeros((CH, 64), F32)

    SA = _sc_stage1(eq[0], eq[1], eq[2], eq[3], rh0, rh1, time_emb,
                    ones_t, zeros_t, srcP, dstP, brelP, timeP, invP)

    bias3 = jnp.concatenate(
        [W_I_b[None], W_O_b[None], W_S_b[None], jnp.zeros((5, ENT), F32)], 0)
    ent_new = _tc_stage1(SA, ent_emb, W_I_w.T, W_O_w.T, W_S_w.T, bias3)

    zeros_t2 = jnp.zeros((104, ENT), F32)
    outS, outD, outT, outC = _sc_stage2(
        ent_new, time_emb, ones_t, zeros_t2, srcP, dstP, brelP, timeP, invP)

    bias3r = jnp.concatenate(
        [W_r_ori_b[None], W_r_inv_b[None], W_R_b[None], jnp.zeros((5, REL), F32)], 0)
    rel_new = _tc_stage2(outS, outD, outT, outC, rel_emb,
                         W_r_ori_w.T, W_r_inv_w.T, W_R_w.T, bias3r)
    return ent_new, rel_new


# speedup vs baseline: 2.5602x; 1.4463x over previous
import functools

import jax
import jax.numpy as jnp
from jax import lax
from jax.experimental import pallas as pl
from jax.experimental.pallas import tpu as pltpu
from jax.experimental.pallas import tpu_sc as plsc

N = 10000
E = 160000
R = 200
ENT = 256
REL = 128
TIME = 64
IN_MSG = TIME + REL + ENT

NC = 2
NS = 16
CH = 128
EP = 163840
NCH = EP // CH
KA = 20480
KB = 416
K = 4
F32 = jnp.float32
NPLANES = 8


def _sc_stage1(eq0, eq1, eq2, eq3, rh0, rh1, t64, ones_t, zeros_t,
               srcR, brelR, timeR, keyR):
    mesh = plsc.VectorSubcoreMesh(core_axis_name="c", subcore_axis_name="s")
    nch = NCH // NS

    @functools.partial(
        pl.kernel,
        out_type=jax.ShapeDtypeStruct((NPLANES, KA, 64), F32),
        mesh=mesh,
        compiler_params=pltpu.CompilerParams(use_tc_tiling_on_sc=False),
        scratch_types=[
            pltpu.VMEM_SHARED((KA, 64), F32),
            pltpu.VMEM((K, CH), jnp.int32),
            pltpu.VMEM((K, CH), jnp.int32),
            pltpu.VMEM((K, CH, 64), F32),
            pltpu.VMEM((CH, 64), F32),
            pltpu.SemaphoreType.DMA,
            pltpu.SemaphoreType.DMA,
            pltpu.SemaphoreType.DMA,
            pltpu.SemaphoreType.DMA,
            pltpu.SemaphoreType.DMA,
        ],
    )
    def k(eq0_h, eq1_h, eq2_h, eq3_h, rh0_h, rh1_h, t64_h, ones_h, zeros_h,
          src_h, brel_h, time_h, key_h, out_h,
          acc, idx4, key4, rows, rones,
          sg0, sg1, sg2, sg3, ss):
        core = lax.axis_index("c")
        s = lax.axis_index("s")
        sgs = [sg0, sg1, sg2, sg3]
        pltpu.sync_copy(ones_h, rones)

        def run_pass(cid, table, idx_h):
            @pl.when(core == cid)
            def _():
                def group(g, carry):
                    base = s * nch + g * K
                    pltpu.sync_copy(idx_h.at[pl.ds(base, K)], idx4)
                    pltpu.sync_copy(key_h.at[pl.ds(base, K)], key4)
                    gds = [
                        pltpu.async_copy(table.at[idx4.at[b]],
                                         rows.at[b], sgs[b])
                        for b in range(K)
                    ]
                    sds = []
                    for b in range(K):
                        gds[b].wait()
                        sds.append(pltpu.async_copy(
                            rows.at[b], acc.at[key4.at[b]], ss,
                            add=True))
                    for b in range(K):
                        sds[b].wait()
                    return carry
                lax.fori_loop(0, nch // K, group, 0)

        def run_count_pass(cid):
            @pl.when(core == cid)
            def _():
                def group(g, carry):
                    base = s * nch + g * K
                    pltpu.sync_copy(key_h.at[pl.ds(base, K)], key4)
                    sds = [
                        pltpu.async_copy(rones, acc.at[key4.at[b]],
                                         ss, add=True)
                        for b in range(K)
                    ]
                    for b in range(K):
                        sds[b].wait()
                    return carry
                lax.fori_loop(0, nch // K, group, 0)

        def flush(cid, plane):
            @pl.when(core == cid)
            def _():
                nr = KA // NS
                pltpu.sync_copy(acc.at[pl.ds(s * nr, nr)],
                                out_h.at[plane, pl.ds(s * nr, nr)])

        rounds = [
            ((eq0_h, src_h, 2), (eq1_h, src_h, 3)),
            ((eq2_h, src_h, 4), (eq3_h, src_h, 5)),
            ((rh0_h, brel_h, 0), (rh1_h, brel_h, 1)),
            ((t64_h, time_h, 6), (None, None, 7)),
        ]
        for p0, p1 in rounds:
            pltpu.sync_copy(zeros_h, acc.at[pl.ds(s * (KA // NS), KA // NS)])
            plsc.subcore_barrier()
            run_pass(0, p0[0], p0[1])
            if p1[0] is None:
                run_count_pass(1)
            else:
                run_pass(1, p1[0], p1[1])
            plsc.subcore_barrier()
            flush(0, p0[2])
            flush(1, p1[2])
            plsc.subcore_barrier()

    return k(eq0, eq1, eq2, eq3, rh0, rh1, t64, ones_t, zeros_t,
             srcR, brelR, timeR, keyR)


def _sc_stage2(ent_new, t64, ones_t, zerosS, zerosT,
               srcR, dstR, timeR, keyR2):
    mesh = plsc.VectorSubcoreMesh(core_axis_name="c", subcore_axis_name="s")
    nch = NCH // (NC * NS)

    @functools.partial(
        pl.kernel,
        out_type=(
            jax.ShapeDtypeStruct((NC, KB, ENT), F32),
            jax.ShapeDtypeStruct((NC, KB, ENT), F32),
            jax.ShapeDtypeStruct((NC, KB, 64), F32),
            jax.ShapeDtypeStruct((NC, KB, 64), F32),
        ),
        mesh=mesh,
        compiler_params=pltpu.CompilerParams(use_tc_tiling_on_sc=False),
        scratch_types=[
            pltpu.VMEM_SHARED((KB, ENT), F32),
            pltpu.VMEM_SHARED((KB, ENT), F32),
            pltpu.VMEM_SHARED((KB, 64), F32),
            pltpu.VMEM_SHARED((KB, 64), F32),
            pltpu.VMEM((nch, CH), jnp.int32),
            pltpu.VMEM((nch, CH), jnp.int32),
            pltpu.VMEM((nch, CH), jnp.int32),
            pltpu.VMEM((nch, CH), jnp.int32),
            pltpu.VMEM((CH, ENT), F32),
            pltpu.VMEM((CH, ENT), F32),
            pltpu.VMEM((CH, 64), F32),
            pltpu.VMEM((CH, 64), F32),
            pltpu.SemaphoreType.DMA,
            pltpu.SemaphoreType.DMA,
            pltpu.SemaphoreType.DMA,
            pltpu.SemaphoreType.DMA,
        ],
    )
    def k(ent_h, t64_h, ones_h, zS_h, zT_h, src_h, dst_h, time_h, key_h,
          outS_h, outD_h, outT_h, outC_h,
          accS, accD, accT, accC, src2d, dst2d, time2d, key2d,
          rowsS, rowsD, rowsT, rones, se0, se1, st0, ss):
        core = lax.axis_index("c")
        s = lax.axis_index("s")
        pltpu.sync_copy(ones_h, rones)
        wid = s * NC + core
        r0 = wid * nch
        pltpu.sync_copy(src_h.at[pl.ds(r0, nch)], src2d)
        pltpu.sync_copy(dst_h.at[pl.ds(r0, nch)], dst2d)
        pltpu.sync_copy(time_h.at[pl.ds(r0, nch)], time2d)
        pltpu.sync_copy(key_h.at[pl.ds(r0, nch)], key2d)

        @pl.when(s == 0)
        def _():
            pltpu.sync_copy(zS_h, accS)

        @pl.when(s == 1)
        def _():
            pltpu.sync_copy(zS_h, accD)

        @pl.when(s == 2)
        def _():
            pltpu.sync_copy(zT_h, accT)

        @pl.when(s == 3)
        def _():
            pltpu.sync_copy(zT_h, accC)

        plsc.subcore_barrier()

        def body(i, carry):
            key = key2d.at[i]
            g0 = pltpu.async_copy(ent_h.at[src2d.at[i]], rowsS, se0)
            g1 = pltpu.async_copy(ent_h.at[dst2d.at[i]], rowsD, se1)
            g2 = pltpu.async_copy(t64_h.at[time2d.at[i]], rowsT, st0)
            g0.wait()
            s0 = pltpu.async_copy(rowsS, accS.at[key], ss, add=True)
            g1.wait()
            s1 = pltpu.async_copy(rowsD, accD.at[key], ss, add=True)
            g2.wait()
            s2 = pltpu.async_copy(rowsT, accT.at[key], ss, add=True)
            s3 = pltpu.async_copy(rones, accC.at[key], ss, add=True)
            s0.wait()
            s1.wait()
            s2.wait()
            s3.wait()
            return carry

        lax.fori_loop(0, nch, body, 0)
        plsc.subcore_barrier()

        @pl.when(s < 13)
        def _():
            nr = 32
            f0 = s * nr
            pltpu.sync_copy(accS.at[pl.ds(f0, nr)], outS_h.at[core, pl.ds(f0, nr)])
            pltpu.sync_copy(accD.at[pl.ds(f0, nr)], outD_h.at[core, pl.ds(f0, nr)])
            pltpu.sync_copy(accT.at[pl.ds(f0, nr)], outT_h.at[core, pl.ds(f0, nr)])
            pltpu.sync_copy(accC.at[pl.ds(f0, nr)], outC_h.at[core, pl.ds(f0, nr)])

    return k(ent_new, t64, ones_t, zerosS, zerosT, srcR, dstR, timeR, keyR2)


def _tc_stage1(SA, ent_emb, wIt, wOt, wSt, bias3):
    BM = 1000
    nb = N // BM

    def body(s0_ref, s1_ref, e_ref, wI_ref, wO_ref, wS_ref, b_ref, o_ref):
        dot = functools.partial(jnp.dot, preferred_element_type=F32,
                                precision=lax.Precision.HIGHEST)
        blk0 = s0_ref[...]
        blk1 = s1_ref[...]
        s0 = jnp.concatenate([blk0[p] for p in range(7)], axis=1)
        d0 = blk0[7][:, 0:1]
        s1 = jnp.concatenate([blk1[p] for p in range(7)], axis=1)
        d1 = blk1[7][:, 0:1]
        m = (dot(s0, wI_ref[...]) + d0 * b_ref[0:1, :]
             + dot(s1, wO_ref[...]) + d1 * b_ref[1:2, :])
        h = m / jnp.maximum(d0 + d1, 1.0)
        o_ref[...] = dot(e_ref[...], wS_ref[...]) + b_ref[2:3, :] + h

    return pl.pallas_call(
        body,
        grid=(nb,),
        in_specs=[
            pl.BlockSpec((NPLANES, BM, 64), lambda i: (0, i, 0)),
            pl.BlockSpec((NPLANES, BM, 64), lambda i: (0, i + nb, 0)),
            pl.BlockSpec((BM, ENT), lambda i: (i, 0)),
            pl.BlockSpec((IN_MSG, ENT), lambda i: (0, 0)),
            pl.BlockSpec((IN_MSG, ENT), lambda i: (0, 0)),
            pl.BlockSpec((ENT, ENT), lambda i: (0, 0)),
            pl.BlockSpec((8, ENT), lambda i: (0, 0)),
        ],
        out_specs=pl.BlockSpec((BM, ENT), lambda i: (i, 0)),
        out_shape=jax.ShapeDtypeStruct((N, ENT), F32),
    )(SA, SA, ent_emb, wIt, wOt, wSt, bias3)


def _tc_stage2(outS, outD, outT, outC, rel_emb, wot, wit, wrt, bias3r):
    def body(S_ref, D_ref, T_ref, C_ref, rel_ref, wo_ref, wi_ref, wr_ref,
             b_ref, o_ref):
        dot = functools.partial(jnp.dot, preferred_element_type=F32,
                                precision=lax.Precision.HIGHEST)
        US = S_ref[0] + S_ref[1]
        UD = D_ref[0] + D_ref[1]
        UT = T_ref[0] + T_ref[1]
        Cc = C_ref[0] + C_ref[1]
        c = Cc[:, 0:1]
        p0 = (dot(US[0:R], wo_ref[0:ENT]) + dot(UD[0:R], wo_ref[ENT:2 * ENT])
              + dot(UT[0:R], wo_ref[2 * ENT:2 * ENT + TIME])
              + c[0:R] * b_ref[0:1, :])
        p1 = (dot(US[R:2 * R], wi_ref[0:ENT])
              + dot(UD[R:2 * R], wi_ref[ENT:2 * ENT])
              + dot(UT[R:2 * R], wi_ref[2 * ENT:2 * ENT + TIME])
              + c[R:2 * R] * b_ref[1:2, :])
        cnt = c[0:R] + c[R:2 * R]
        h = (p0 + p1) / jnp.maximum(cnt, 1.0)
        o_ref[...] = dot(rel_ref[...], wr_ref[...]) + b_ref[2:3, :] + h

    return pl.pallas_call(
        body,
        out_shape=jax.ShapeDtypeStruct((R, REL), F32),
    )(outS, outD, outT, outC, rel_emb, wot, wit, wrt, bias3r)


def kernel(ent_emb, rel_emb, time_emb, edge_index, b_rel, time_idx, inv,
           W_I_w, W_I_b, W_O_w, W_O_b, W_S_w, W_S_b,
           W_r_ori_w, W_r_ori_b, W_r_inv_w, W_r_inv_b, W_R_w, W_R_b):
    i32 = jnp.int32
    pad = EP - E
    src = edge_index[0].astype(i32)
    dst = edge_index[1].astype(i32)
    zpad = jnp.zeros((pad,), i32)
    srcP = jnp.concatenate([src, zpad])
    dstP = jnp.concatenate([dst, zpad])
    brelP = jnp.concatenate([b_rel.astype(i32), zpad])
    timeP = jnp.concatenate([time_idx.astype(i32), zpad])
    invP = jnp.concatenate([inv.astype(i32), jnp.full((pad,), 2, i32)])
    srcR = srcP.reshape(NCH, CH)
    dstR = dstP.reshape(NCH, CH)
    brelR = brelP.reshape(NCH, CH)
    timeR = timeP.reshape(NCH, CH)
    keyR = (invP * N + dstP).reshape(NCH, CH)
    keyR2 = (invP * R + brelP).reshape(NCH, CH)

    eq = [ent_emb[:, 64 * k:64 * (k + 1)] for k in range(4)]
    rh0 = rel_emb[:, :64]
    rh1 = rel_emb[:, 64:]
    ones_t = jnp.ones((CH, 64), F32)
    zeros_t = jnp.zeros((KA // NS, 64), F32)

    SA = _sc_stage1(eq[0], eq[1], eq[2], eq[3], rh0, rh1, time_emb,
                    ones_t, zeros_t, srcR, brelR, timeR, keyR)

    bias3 = jnp.concatenate(
        [W_I_b[None], W_O_b[None], W_S_b[None], jnp.zeros((5, ENT), F32)], 0)
    ent_new = _tc_stage1(SA, ent_emb, W_I_w.T, W_O_w.T, W_S_w.T, bias3)

    zerosS = jnp.zeros((KB, ENT), F32)
    zerosT = jnp.zeros((KB, 64), F32)
    outS, outD, outT, outC = _sc_stage2(
        ent_new, time_emb, ones_t, zerosS, zerosT, srcR, dstR, timeR, keyR2)

    bias3r = jnp.concatenate(
        [W_r_ori_b[None], W_r_inv_b[None], W_R_b[None], jnp.zeros((5, REL), F32)], 0)
    rel_new = _tc_stage2(outS, outD, outT, outC, rel_emb,
                         W_r_ori_w.T, W_r_inv_w.T, W_R_w.T, bias3r)
    return ent_new, rel_new


# speedup vs baseline: 2.8991x; 1.1324x over previous
import functools

import jax
import jax.numpy as jnp
from jax import lax
from jax.experimental import pallas as pl
from jax.experimental.pallas import tpu as pltpu
from jax.experimental.pallas import tpu_sc as plsc

N = 10000
E = 160000
R = 200
ENT = 256
REL = 128
TIME = 64
IN_MSG = TIME + REL + ENT

NC = 2
NS = 16
CH = 128
CH2 = 64
EP = 163840
KA = 20008
KB = 416
K = 4
F32 = jnp.float32
NPLANES = 8


def _sc_stage1(eq0, eq1, eq2, eq3, rh0, rh1, t64, ones_t, zeros_t,
               srcR, brelR, timeR, keyR):
    mesh = plsc.VectorSubcoreMesh(core_axis_name="c", subcore_axis_name="s")
    nch = EP // CH // NS
    half = nch // 2

    @functools.partial(
        pl.kernel,
        out_type=jax.ShapeDtypeStruct((NPLANES, KA, 64), F32),
        mesh=mesh,
        compiler_params=pltpu.CompilerParams(use_tc_tiling_on_sc=False),
        scratch_types=[
            pltpu.VMEM_SHARED((KA, 64), F32),
            pltpu.VMEM((half, CH), jnp.int32),
            pltpu.VMEM((nch, CH), jnp.int32),
            pltpu.VMEM((K, CH, 64), F32),
            pltpu.SemaphoreType.DMA,
            pltpu.SemaphoreType.DMA,
            pltpu.SemaphoreType.DMA,
            pltpu.SemaphoreType.DMA,
            pltpu.SemaphoreType.DMA,
            pltpu.SemaphoreType.DMA,
            pltpu.SemaphoreType.DMA,
            pltpu.SemaphoreType.DMA,
        ],
    )
    def k(eq0_h, eq1_h, eq2_h, eq3_h, rh0_h, rh1_h, t64_h, ones_h, zeros_h,
          src_h, brel_h, time_h, key_h, out_h,
          acc, idx2d, key2d, rows,
          sg0, sg1, sg2, sg3, ss0, ss1, ss2, ss3):
        core = lax.axis_index("c")
        s = lax.axis_index("s")
        sg = [sg0, sg1, sg2, sg3]
        ss = [ss0, ss1, ss2, ss3]
        pltpu.sync_copy(key_h.at[pl.ds(s * nch, nch)], key2d)

        def wait_gather(table, b):
            pltpu.make_async_copy(table.at[idx2d.at[0]], rows.at[b],
                                  sg[b]).wait()

        def wait_scatter(b):
            pltpu.make_async_copy(rows.at[b], acc.at[key2d.at[0]],
                                  ss[b]).wait()

        def run_pass(cid, table, idx_h):
            @pl.when(core == cid)
            def _():
                for h in range(2):
                    pltpu.sync_copy(
                        idx_h.at[pl.ds(s * nch + h * half, half)], idx2d)
                    for b in range(K):
                        pltpu.async_copy(table.at[idx2d.at[b]], rows.at[b],
                                         sg[b])

                    def it(t, carry, h=h):
                        for b in range(K):
                            li = t * K + b
                            wait_gather(table, b)
                            pltpu.async_copy(
                                rows.at[b], acc.at[key2d.at[h * half + li]],
                                ss[b], add=True)
                        for b in range(K):
                            nli = t * K + K + b

                            @pl.when(nli < half)
                            def _(nli=nli, b=b):
                                wait_scatter(b)
                                pltpu.async_copy(table.at[idx2d.at[nli]],
                                                 rows.at[b], sg[b])
                        return carry

                    lax.fori_loop(0, half // K, it, 0)
                    for b in range(K):
                        wait_scatter(b)

        def run_count_pass(cid):
            @pl.when(core == cid)
            def _():
                pltpu.sync_copy(ones_h, rows.at[0])

                def it(t, carry):
                    ds_ = [
                        pltpu.async_copy(rows.at[0],
                                         acc.at[key2d.at[t * K + b]],
                                         ss[b], add=True)
                        for b in range(K)
                    ]
                    for d in ds_:
                        d.wait()
                    return carry

                lax.fori_loop(0, nch // K, it, 0)

        def flush(cid, plane):
            @pl.when(core == cid)
            def _():
                @pl.when(s < 15)
                def _():
                    pltpu.sync_copy(acc.at[pl.ds(s * 1256, 1256)],
                                    out_h.at[plane, pl.ds(s * 1256, 1256)])

                @pl.when(s == 15)
                def _():
                    pltpu.sync_copy(acc.at[pl.ds(18840, 1168)],
                                    out_h.at[plane, pl.ds(18840, 1168)])

        def zero_acc():
            @pl.when(s < 15)
            def _():
                pltpu.sync_copy(zeros_h, acc.at[pl.ds(s * 1256, 1256)])

            @pl.when(s == 15)
            def _():
                pltpu.sync_copy(zeros_h.at[pl.ds(0, 1168)],
                                acc.at[pl.ds(18840, 1168)])

        rounds = [
            ((eq0_h, src_h, 2), (eq1_h, src_h, 3)),
            ((eq2_h, src_h, 4), (eq3_h, src_h, 5)),
            ((rh0_h, brel_h, 0), (rh1_h, brel_h, 1)),
            ((t64_h, time_h, 6), (None, None, 7)),
        ]
        for p0, p1 in rounds:
            zero_acc()
            plsc.subcore_barrier()
            run_pass(0, p0[0], p0[1])
            if p1[0] is None:
                run_count_pass(1)
            else:
                run_pass(1, p1[0], p1[1])
            plsc.subcore_barrier()
            flush(0, p0[2])
            flush(1, p1[2])
            plsc.subcore_barrier()

    return k(eq0, eq1, eq2, eq3, rh0, rh1, t64, ones_t, zeros_t,
             srcR, brelR, timeR, keyR)


def _sc_stage2(ent_new, t64, ones_t, zerosS, zerosT,
               srcR2, dstR2, timeR2, keyR2):
    mesh = plsc.VectorSubcoreMesh(core_axis_name="c", subcore_axis_name="s")
    nch = EP // CH2 // (NC * NS)

    @functools.partial(
        pl.kernel,
        out_type=(
            jax.ShapeDtypeStruct((NC, KB, ENT), F32),
            jax.ShapeDtypeStruct((NC, KB, ENT), F32),
            jax.ShapeDtypeStruct((NC, KB, 64), F32),
            jax.ShapeDtypeStruct((NC, KB, 64), F32),
        ),
        mesh=mesh,
        compiler_params=pltpu.CompilerParams(use_tc_tiling_on_sc=False),
        scratch_types=[
            pltpu.VMEM_SHARED((KB, ENT), F32),
            pltpu.VMEM_SHARED((KB, ENT), F32),
            pltpu.VMEM_SHARED((KB, 64), F32),
            pltpu.VMEM_SHARED((KB, 64), F32),
            pltpu.VMEM((nch, CH2), jnp.int32),
            pltpu.VMEM((nch, CH2), jnp.int32),
            pltpu.VMEM((nch, CH2), jnp.int32),
            pltpu.VMEM((nch, CH2), jnp.int32),
            pltpu.VMEM((2, CH2, ENT), F32),
            pltpu.VMEM((2, CH2, ENT), F32),
            pltpu.VMEM((2, CH2, 64), F32),
            pltpu.VMEM((CH2, 64), F32),
            pltpu.SemaphoreType.DMA,
            pltpu.SemaphoreType.DMA,
            pltpu.SemaphoreType.DMA,
            pltpu.SemaphoreType.DMA,
            pltpu.SemaphoreType.DMA,
            pltpu.SemaphoreType.DMA,
            pltpu.SemaphoreType.DMA,
            pltpu.SemaphoreType.DMA,
            pltpu.SemaphoreType.DMA,
            pltpu.SemaphoreType.DMA,
            pltpu.SemaphoreType.DMA,
            pltpu.SemaphoreType.DMA,
            pltpu.SemaphoreType.DMA,
            pltpu.SemaphoreType.DMA,
        ],
    )
    def k(ent_h, t64_h, ones_h, zS_h, zT_h, src_h, dst_h, time_h, key_h,
          outS_h, outD_h, outT_h, outC_h,
          accS, accD, accT, accC, src2d, dst2d, time2d, key2d,
          rs, rd, rt, rones,
          gs0, gs1, gd0, gd1, gt0, gt1,
          ws0, ws1, wd0, wd1, wt0, wt1, wc0, wc1):
        core = lax.axis_index("c")
        s = lax.axis_index("s")
        gs = [gs0, gs1]
        gd = [gd0, gd1]
        gt = [gt0, gt1]
        ws = [ws0, ws1]
        wd = [wd0, wd1]
        wt = [wt0, wt1]
        wc = [wc0, wc1]
        pltpu.sync_copy(ones_h, rones)
        wid = s * NC + core
        r0 = wid * nch
        pltpu.sync_copy(src_h.at[pl.ds(r0, nch)], src2d)
        pltpu.sync_copy(dst_h.at[pl.ds(r0, nch)], dst2d)
        pltpu.sync_copy(time_h.at[pl.ds(r0, nch)], time2d)
        pltpu.sync_copy(key_h.at[pl.ds(r0, nch)], key2d)

        @pl.when(s == 0)
        def _():
            pltpu.sync_copy(zS_h, accS)

        @pl.when(s == 1)
        def _():
            pltpu.sync_copy(zS_h, accD)

        @pl.when(s == 2)
        def _():
            pltpu.sync_copy(zT_h, accT)

        @pl.when(s == 3)
        def _():
            pltpu.sync_copy(zT_h, accC)

        plsc.subcore_barrier()

        def issue_gathers(i, sl):
            pltpu.async_copy(ent_h.at[src2d.at[i]], rs.at[sl], gs[sl])
            pltpu.async_copy(ent_h.at[dst2d.at[i]], rd.at[sl], gd[sl])
            pltpu.async_copy(t64_h.at[time2d.at[i]], rt.at[sl], gt[sl])

        def wait_gathers(sl):
            pltpu.make_async_copy(ent_h.at[src2d.at[0]], rs.at[sl], gs[sl]).wait()
            pltpu.make_async_copy(ent_h.at[dst2d.at[0]], rd.at[sl], gd[sl]).wait()
            pltpu.make_async_copy(t64_h.at[time2d.at[0]], rt.at[sl], gt[sl]).wait()

        def issue_scatters(i, sl):
            key = key2d.at[i]
            pltpu.async_copy(rs.at[sl], accS.at[key], ws[sl], add=True)
            pltpu.async_copy(rd.at[sl], accD.at[key], wd[sl], add=True)
            pltpu.async_copy(rt.at[sl], accT.at[key], wt[sl], add=True)
            pltpu.async_copy(rones, accC.at[key], wc[sl], add=True)

        def wait_scatters(sl):
            pltpu.make_async_copy(rs.at[sl], accS.at[key2d.at[0]], ws[sl]).wait()
            pltpu.make_async_copy(rd.at[sl], accD.at[key2d.at[0]], wd[sl]).wait()
            pltpu.make_async_copy(rt.at[sl], accT.at[key2d.at[0]], wt[sl]).wait()
            pltpu.make_async_copy(rones, accC.at[key2d.at[0]], wc[sl]).wait()

        issue_gathers(0, 0)
        issue_gathers(1, 1)

        def it(u, carry):
            for sl in range(2):
                i = 2 * u + sl
                wait_gathers(sl)
                issue_scatters(i, sl)
            for sl in range(2):
                ni = 2 * u + 2 + sl

                @pl.when(ni < nch)
                def _(ni=ni, sl=sl):
                    wait_scatters(sl)
                    issue_gathers(ni, sl)
            return carry

        lax.fori_loop(0, nch // 2, it, 0)
        wait_scatters(0)
        wait_scatters(1)
        plsc.subcore_barrier()

        @pl.when(s < 13)
        def _():
            nr = 32
            f0 = s * nr
            pltpu.sync_copy(accS.at[pl.ds(f0, nr)], outS_h.at[core, pl.ds(f0, nr)])
            pltpu.sync_copy(accD.at[pl.ds(f0, nr)], outD_h.at[core, pl.ds(f0, nr)])
            pltpu.sync_copy(accT.at[pl.ds(f0, nr)], outT_h.at[core, pl.ds(f0, nr)])
            pltpu.sync_copy(accC.at[pl.ds(f0, nr)], outC_h.at[core, pl.ds(f0, nr)])

    return k(ent_new, t64, ones_t, zerosS, zerosT, srcR2, dstR2, timeR2, keyR2)


def _tc_stage1(SA, ent_emb, wIt, wOt, wSt, bias3):
    BM = 1000
    nb = N // BM

    def body(s0_ref, s1_ref, e_ref, wI_ref, wO_ref, wS_ref, b_ref, o_ref):
        dot = functools.partial(jnp.dot, preferred_element_type=F32,
                                precision=lax.Precision.HIGHEST)
        blk0 = s0_ref[...]
        blk1 = s1_ref[...]
        s0 = jnp.concatenate([blk0[p] for p in range(7)], axis=1)
        d0 = blk0[7][:, 0:1]
        s1 = jnp.concatenate([blk1[p] for p in range(7)], axis=1)
        d1 = blk1[7][:, 0:1]
        m = (dot(s0, wI_ref[...]) + d0 * b_ref[0:1, :]
             + dot(s1, wO_ref[...]) + d1 * b_ref[1:2, :])
        h = m / jnp.maximum(d0 + d1, 1.0)
        o_ref[...] = dot(e_ref[...], wS_ref[...]) + b_ref[2:3, :] + h

    return pl.pallas_call(
        body,
        grid=(nb,),
        in_specs=[
            pl.BlockSpec((NPLANES, BM, 64), lambda i: (0, i, 0)),
            pl.BlockSpec((NPLANES, BM, 64), lambda i: (0, i + nb, 0)),
            pl.BlockSpec((BM, ENT), lambda i: (i, 0)),
            pl.BlockSpec((IN_MSG, ENT), lambda i: (0, 0)),
            pl.BlockSpec((IN_MSG, ENT), lambda i: (0, 0)),
            pl.BlockSpec((ENT, ENT), lambda i: (0, 0)),
            pl.BlockSpec((8, ENT), lambda i: (0, 0)),
        ],
        out_specs=pl.BlockSpec((BM, ENT), lambda i: (i, 0)),
        out_shape=jax.ShapeDtypeStruct((N, ENT), F32),
    )(SA, SA, ent_emb, wIt, wOt, wSt, bias3)


def _tc_stage2(outS, outD, outT, outC, rel_emb, wot, wit, wrt, bias3r):
    def body(S_ref, D_ref, T_ref, C_ref, rel_ref, wo_ref, wi_ref, wr_ref,
             b_ref, o_ref):
        dot = functools.partial(jnp.dot, preferred_element_type=F32,
                                precision=lax.Precision.HIGHEST)
        US = S_ref[0] + S_ref[1]
        UD = D_ref[0] + D_ref[1]
        UT = T_ref[0] + T_ref[1]
        Cc = C_ref[0] + C_ref[1]
        c = Cc[:, 0:1]
        p0 = (dot(US[0:R], wo_ref[0:ENT]) + dot(UD[0:R], wo_ref[ENT:2 * ENT])
              + dot(UT[0:R], wo_ref[2 * ENT:2 * ENT + TIME])
              + c[0:R] * b_ref[0:1, :])
        p1 = (dot(US[R:2 * R], wi_ref[0:ENT])
              + dot(UD[R:2 * R], wi_ref[ENT:2 * ENT])
              + dot(UT[R:2 * R], wi_ref[2 * ENT:2 * ENT + TIME])
              + c[R:2 * R] * b_ref[1:2, :])
        cnt = c[0:R] + c[R:2 * R]
        h = (p0 + p1) / jnp.maximum(cnt, 1.0)
        o_ref[...] = dot(rel_ref[...], wr_ref[...]) + b_ref[2:3, :] + h

    return pl.pallas_call(
        body,
        out_shape=jax.ShapeDtypeStruct((R, REL), F32),
    )(outS, outD, outT, outC, rel_emb, wot, wit, wrt, bias3r)


def kernel(ent_emb, rel_emb, time_emb, edge_index, b_rel, time_idx, inv,
           W_I_w, W_I_b, W_O_w, W_O_b, W_S_w, W_S_b,
           W_r_ori_w, W_r_ori_b, W_r_inv_w, W_r_inv_b, W_R_w, W_R_b):
    i32 = jnp.int32
    pad = EP - E
    src = edge_index[0].astype(i32)
    dst = edge_index[1].astype(i32)
    zpad = jnp.zeros((pad,), i32)
    srcP = jnp.concatenate([src, zpad])
    dstP = jnp.concatenate([dst, zpad])
    brelP = jnp.concatenate([b_rel.astype(i32), zpad])
    timeP = jnp.concatenate([time_idx.astype(i32), zpad])
    invP = jnp.concatenate([inv.astype(i32), jnp.full((pad,), 2, i32)])
    nchT = EP // CH
    nchT2 = EP // CH2
    srcR = srcP.reshape(nchT, CH)
    brelR = brelP.reshape(nchT, CH)
    timeR = timeP.reshape(nchT, CH)
    keyR = (invP * N + dstP).reshape(nchT, CH)
    srcR2 = srcP.reshape(nchT2, CH2)
    dstR2 = dstP.reshape(nchT2, CH2)
    timeR2 = timeP.reshape(nchT2, CH2)
    keyR2 = (invP * R + brelP).reshape(nchT2, CH2)

    eq = [ent_emb[:, 64 * k:64 * (k + 1)] for k in range(4)]
    rh0 = rel_emb[:, :64]
    rh1 = rel_emb[:, 64:]
    ones_t = jnp.ones((CH, 64), F32)
    zeros_t = jnp.zeros((1256, 64), F32)

    SA = _sc_stage1(eq[0], eq[1], eq[2], eq[3], rh0, rh1, time_emb,
                    ones_t, zeros_t, srcR, brelR, timeR, keyR)

    bias3 = jnp.concatenate(
        [W_I_b[None], W_O_b[None], W_S_b[None], jnp.zeros((5, ENT), F32)], 0)
    ent_new = _tc_stage1(SA, ent_emb, W_I_w.T, W_O_w.T, W_S_w.T, bias3)

    ones_t2 = jnp.ones((CH2, 64), F32)
    zerosS = jnp.zeros((KB, ENT), F32)
    zerosT = jnp.zeros((KB, 64), F32)
    outS, outD, outT, outC = _sc_stage2(
        ent_new, time_emb, ones_t2, zerosS, zerosT,
        srcR2, dstR2, timeR2, keyR2)

    bias3r = jnp.concatenate(
        [W_r_ori_b[None], W_r_inv_b[None], W_R_b[None], jnp.zeros((5, REL), F32)], 0)
    rel_new = _tc_stage2(outS, outD, outT, outC, rel_emb,
                         W_r_ori_w.T, W_r_inv_w.T, W_R_w.T, bias3r)
    return ent_new, rel_new
